# Initial kernel scaffold; baseline (speedup 1.0000x reference)
import jax
import jax.numpy as jnp
from jax.experimental import pallas as pl


def kernel(x_inp_core, edge_index_core, edge_attr_core, x_upd_core, Z_core, Z_block, node2graph_core, W_embed, b_embed, W_edge, b_edge, W_msg, b_msg, W1, b1, W2, b2):
    raise NotImplementedError("write your pallas kernel here")


# speedup vs baseline: 1.5521x; 1.5521x over previous
import functools
import jax
import jax.numpy as jnp
from jax import lax
from jax.experimental import pallas as pl
from jax.experimental.pallas import tpu as pltpu
from jax.experimental.pallas import tpu_sc as plsc

V = 10000
E = 320000
G = 128
D = 128
DE = 16
NL = 4

NC = 2
NS = 16
NW = NC * NS
CH = 80
NBT = E // CH
VP = 10240
ROWS_PER_TILE = VP // NS
ZR = 40
MAXRUN = 128
DUMMY = VP - 1


def _chunk_sizes():
  sizes = []
  for _half in range(2):
    r = E // 2
    for i in range(NS):
      c = r if i == NS - 1 else -(-r // (NS - i) // 240) * 240
      sizes.append(c)
      r -= c
  return sizes


_CS = _chunk_sizes()
_CB = [0]
for _c in _CS:
  _CB.append(_CB[-1] + _c)
_BOUNDS = _CB[1:-1]
_PER_SC_BATCHES = sum(c // CH for c in _CS[:NS])

_mesh = plsc.VectorSubcoreMesh(
    core_axis_name="c", subcore_axis_name="s", num_cores=NC, num_subcores=NS)


def _sc_edge_body(h_hbm, e_hbm, src3_hbm, eid3_hbm, dst3_hbm, bsrcx_hbm,
                  beidx_hbm, pidx_hbm, dense_hbm,
                  srcv, dstv, eidv, hrow, erow, zero_v, idxsub, idxsub2,
                  bh, be, patchv, idxv16, agg_sh, sem, sem2):
  c = lax.axis_index("c")
  s = lax.axis_index("s")
  wid = s * NC + c

  @pl.loop(0, ZR)
  def _zero(r):
    for k in range(D // 16):
      zero_v[r, pl.ds(k * 16, 16)] = jnp.zeros((16,), jnp.float32)

  for rep in range(ROWS_PER_TILE // ZR):
    pltpu.sync_copy(zero_v,
                    agg_sh.at[pl.ds(s * ROWS_PER_TILE + rep * ZR, ZR)])
  plsc.subcore_barrier()

  nbat = jnp.where(s < 11, 126, jnp.where(s < 15, 123, 122))
  lsb = jnp.where(s < 11, s * 126,
                  jnp.where(s < 15, 1386 + (s - 11) * 123,
                            1878 + (s - 15) * 122))
  base_gb = c * _PER_SC_BATCHES + lsb

  @pl.loop(0, nbat)
  def _batch(j):
    gb = base_gb + j
    pltpu.sync_copy(src3_hbm.at[gb], srcv)
    pltpu.sync_copy(dst3_hbm.at[gb], dstv)
    pltpu.sync_copy(eid3_hbm.at[gb], eidv)
    pltpu.async_copy(h_hbm.at[srcv.at[0]], hrow, sem).wait()
    pltpu.async_copy(e_hbm.at[eidv.at[0]], erow, sem2).wait()

    @pl.loop(0, CH)
    def _row(r):
      for k in range(D // 16):
        sl = pl.ds(k * 16, 16)
        hrow[r, sl] = jnp.maximum(hrow[r, sl] + erow[r, sl], 0.0)

    pltpu.sync_copy(hrow, agg_sh.at[dstv.at[0]], add=True)

  for p in range(2):
    for k in range(D // 16):
      patchv[p, pl.ds(k * 16, 16)] = jnp.zeros((16,), jnp.float32)

  @pl.when(wid < NW - 1)
  def _bnd():
    for p in range(2):
      @pl.loop(0, MAXRUN // 16)
      def _sub(sb):
        flat = (wid * 2 + p) * (MAXRUN // 16) + sb
        pltpu.sync_copy(bsrcx_hbm.at[flat], idxsub)
        pltpu.async_copy(h_hbm.at[idxsub.at[0]], bh, sem).wait()
        pltpu.sync_copy(beidx_hbm.at[flat], idxsub2)
        pltpu.async_copy(e_hbm.at[idxsub2.at[0]], be, sem2).wait()
        for r in range(16):
          for k in range(D // 16):
            sl = pl.ds(k * 16, 16)
            patchv[p, sl] = (patchv[p, sl]
                             + jnp.maximum(bh[r, sl] + be[r, sl], 0.0))

  pltpu.sync_copy(pidx_hbm.at[wid], idxv16)
  for k in range(D // 16):
    sl = pl.ds(k * 16, 16)
    bh[0, sl] = patchv[0, sl] + patchv[1, sl]
  pltpu.sync_copy(bh, agg_sh.at[idxv16.at[0]], add=True)

  plsc.subcore_barrier()
  pltpu.sync_copy(agg_sh.at[pl.ds(s * ROWS_PER_TILE, ROWS_PER_TILE)],
                  dense_hbm.at[c, pl.ds(s * ROWS_PER_TILE, ROWS_PER_TILE)])


_sc_edge_pass = functools.partial(
    pl.kernel,
    out_type=jax.ShapeDtypeStruct((NC, VP, D), jnp.float32),
    mesh=_mesh,
    scratch_types=[
        pltpu.VMEM((1, CH), jnp.int32),
        pltpu.VMEM((1, CH), jnp.int32),
        pltpu.VMEM((1, CH), jnp.int32),
        pltpu.VMEM((CH, D), jnp.float32),
        pltpu.VMEM((CH, D), jnp.float32),
        pltpu.VMEM((ZR, D), jnp.float32),
        pltpu.VMEM((1, 16), jnp.int32),
        pltpu.VMEM((1, 16), jnp.int32),
        pltpu.VMEM((16, D), jnp.float32),
        pltpu.VMEM((16, D), jnp.float32),
        pltpu.VMEM((2, D), jnp.float32),
        pltpu.VMEM((1, 16), jnp.int32),
        pltpu.VMEM_SHARED((VP, D), jnp.float32),
        pltpu.SemaphoreType.DMA,
        pltpu.SemaphoreType.DMA,
    ],
)(_sc_edge_body)


_VB = 1000


def _embed_body(x_ref, n2g_ref, zc_ref, zb_ref, we_ref, b_ref, o_ref):
  oh = (n2g_ref[...] == lax.broadcasted_iota(jnp.int32, (1, G), 1)
        ).astype(jnp.float32)
  g1 = jnp.dot(oh, zc_ref[...], preferred_element_type=jnp.float32)
  g2 = jnp.dot(oh, zb_ref[...], preferred_element_type=jnp.float32)
  xc = jnp.concatenate([x_ref[...], g1, g2], axis=1)
  acc = jnp.dot(xc, we_ref[...], preferred_element_type=jnp.float32)
  o_ref[...] = jnp.maximum(acc + b_ref[...], 0.0)


def _embed_call(x_upd, n2g2d, zc, zb, we, b2d):
  return pl.pallas_call(
      _embed_body,
      grid=(V // _VB,),
      in_specs=[
          pl.BlockSpec((_VB, D), lambda i: (i, 0)),
          pl.BlockSpec((_VB, 1), lambda i: (i, 0)),
          pl.BlockSpec((G, D), lambda i: (0, 0)),
          pl.BlockSpec((G, D), lambda i: (0, 0)),
          pl.BlockSpec((3 * D, D), lambda i: (0, 0)),
          pl.BlockSpec((1, D), lambda i: (0, 0)),
      ],
      out_specs=pl.BlockSpec((_VB, D), lambda i: (i, 0)),
      out_shape=jax.ShapeDtypeStruct((V, D), jnp.float32),
  )(x_upd, n2g2d, zc, zb, we, b2d)


_EB = 2000


def _edge_feat_body(ea_ref, w_ref, b_ref, o_ref):
  acc = jnp.dot(ea_ref[...], w_ref[...], preferred_element_type=jnp.float32)
  o_ref[...] = jnp.maximum(acc + b_ref[...], 0.0)


def _edge_feat_call(edge_attr, w, b2d):
  return pl.pallas_call(
      _edge_feat_body,
      grid=(E // _EB,),
      in_specs=[
          pl.BlockSpec((_EB, DE), lambda i: (i, 0)),
          pl.BlockSpec((DE, D), lambda i: (0, 0)),
          pl.BlockSpec((1, D), lambda i: (0, 0)),
      ],
      out_specs=pl.BlockSpec((_EB, D), lambda i: (i, 0)),
      out_shape=jax.ShapeDtypeStruct((E, D), jnp.float32),
  )(edge_attr, w, b2d)


def _layer_body(h_ref, p0_ref, p1_ref, w_ref, b_ref, o_ref):
  agg = p0_ref[0] + p1_ref[0]
  hc = jnp.concatenate([h_ref[...], agg], axis=1)
  acc = jnp.dot(hc, w_ref[...], preferred_element_type=jnp.float32)
  o_ref[...] = h_ref[...] + jnp.maximum(acc + b_ref[...], 0.0)


def _layer_call(h, dense, wm, b2d):
  return pl.pallas_call(
      _layer_body,
      grid=(V // _VB,),
      in_specs=[
          pl.BlockSpec((_VB, D), lambda i: (i, 0)),
          pl.BlockSpec((1, _VB, D), lambda i: (0, i, 0)),
          pl.BlockSpec((1, _VB, D), lambda i: (1, i, 0)),
          pl.BlockSpec((2 * D, D), lambda i: (0, 0)),
          pl.BlockSpec((1, D), lambda i: (0, 0)),
      ],
      out_specs=pl.BlockSpec((_VB, D), lambda i: (i, 0)),
      out_shape=jax.ShapeDtypeStruct((V, D), jnp.float32),
  )(h, dense, dense, wm, b2d)


def _head_body(h_ref, x_ref, w1_ref, b1_ref, w2_ref, b2_ref,
               n2g_ref, o_ref):
  xc = jnp.concatenate([h_ref[...], x_ref[...]], axis=1)
  hid = jnp.dot(xc, w1_ref[...], preferred_element_type=jnp.float32)
  hid = jnp.maximum(hid + b1_ref[...], 0.0)
  logit = jnp.dot(hid, w2_ref[...],
                  preferred_element_type=jnp.float32) + b2_ref[...]
  oh = (n2g_ref[...] == lax.broadcasted_iota(jnp.int32, (1, G), 1)
        ).astype(jnp.float32)
  masked = jnp.where(oh > 0.0, logit, -1e30)
  mxg = jnp.max(masked, axis=0, keepdims=True)
  mx_node = jnp.dot(oh, jnp.transpose(mxg), preferred_element_type=jnp.float32,
                    precision=lax.Precision.HIGHEST)
  ex = jnp.exp(logit - mx_node)
  den_g = jnp.sum(oh * ex, axis=0, keepdims=True)
  den_node = jnp.dot(oh, jnp.transpose(den_g),
                     preferred_element_type=jnp.float32,
                     precision=lax.Precision.HIGHEST)
  o_ref[...] = ex / den_node


def _head_call(h, x_inp, w1, b1_2d, w2, b2_2d, n2g2d):
  return pl.pallas_call(
      _head_body,
      grid=(1,),
      in_specs=[
          pl.BlockSpec((V, D), lambda i: (0, 0)),
          pl.BlockSpec((V, D), lambda i: (0, 0)),
          pl.BlockSpec((2 * D, D), lambda i: (0, 0)),
          pl.BlockSpec((1, D), lambda i: (0, 0)),
          pl.BlockSpec((D, 1), lambda i: (0, 0)),
          pl.BlockSpec((1, 1), lambda i: (0, 0)),
          pl.BlockSpec((V, 1), lambda i: (0, 0)),
      ],
      out_specs=pl.BlockSpec((V, 1), lambda i: (0, 0)),
      out_shape=jax.ShapeDtypeStruct((V, 1), jnp.float32),
  )(h, x_inp, w1, b1_2d, w2, b2_2d, n2g2d)


def kernel(x_inp_core, edge_index_core, edge_attr_core, x_upd_core, Z_core,
           Z_block, node2graph_core, W_embed, b_embed, W_edge, b_edge, W_msg,
           b_msg, W1, b1, W2, b2):
  n2g2d = node2graph_core.reshape(V, 1)

  dst0 = edge_index_core[1]
  order = jnp.argsort(dst0, stable=True).astype(jnp.int32)
  src_s = edge_index_core[0][order]
  dst_s = dst0[order]
  bnd = jnp.asarray(_BOUNDS, jnp.int32)
  nb = dst_s[bnd]
  lo = jnp.searchsorted(dst_s, nb, side='left').astype(jnp.int32)
  hi = jnp.searchsorted(dst_s, nb, side='right').astype(jnp.int32)
  merged = (lo < bnd) & (hi > bnd)
  lo_m = jnp.where(merged, lo, bnd)
  hi_m = jnp.where(merged, hi, bnd)
  marks = jnp.zeros((E + 1,), jnp.int32)
  marks = marks.at[lo_m].add(1).at[hi_m].add(-1)
  excl = jnp.cumsum(marks[:E]) > 0
  dst_m = jnp.where(excl, jnp.int32(DUMMY), dst_s)

  src3 = src_s.reshape(NBT, 1, CH)
  eid3 = order.reshape(NBT, 1, CH)
  dst3 = dst_m.reshape(NBT, 1, CH)
  a_arr = jnp.stack([lo_m, bnd], axis=1)
  b_arr = jnp.stack([bnd, hi_m], axis=1)
  jj = jnp.arange(MAXRUN, dtype=jnp.int32)
  rows = a_arr[:, :, None] + jj[None, None, :]
  valid = rows < b_arr[:, :, None]
  rowsc = jnp.clip(rows, 0, E - 1)
  bsrcx = jnp.where(valid, src_s[rowsc], jnp.int32(V))
  beidx = jnp.where(valid, order[rowsc], jnp.int32(E))
  padw = jnp.full((1, 2, MAXRUN), V, jnp.int32)
  bsrcx = jnp.concatenate([bsrcx, padw], 0).reshape(NW * 2 * (MAXRUN // 16),
                                                   1, 16)
  beidx = jnp.concatenate([beidx, jnp.full((1, 2, MAXRUN), E, jnp.int32)],
                          0).reshape(NW * 2 * (MAXRUN // 16), 1, 16)
  pidx = jnp.full((NW, 1, 16), DUMMY, jnp.int32)
  pidx = pidx.at[:31, 0, 0].set(jnp.where(merged, nb, DUMMY))

  h = _embed_call(x_upd_core, n2g2d, Z_core, Z_block, W_embed,
                  b_embed.reshape(1, D))
  e = _edge_feat_call(edge_attr_core, W_edge, b_edge.reshape(1, D))

  zpad = jnp.zeros((16, D), jnp.float32)
  ep = jnp.concatenate([e, zpad], 0)
  for l in range(NL):
    hp = jnp.concatenate([h, zpad], 0)
    dense = _sc_edge_pass(hp, ep, src3, eid3, dst3, bsrcx, beidx, pidx)
    h = _layer_call(h, dense, W_msg[l], b_msg[l].reshape(1, D))

  P = _head_call(h, x_inp_core, W1, b1.reshape(1, D), W2,
                 b2.reshape(1, 1), n2g2d)
  return P.reshape(V)
